# baseline (speedup 1.0000x reference)
import functools

import jax
import jax.numpy as jnp
from jax import lax
from jax.experimental import pallas as pl
from jax.experimental.pallas import tpu as pltpu
from jax.experimental.pallas import tpu_sc as plsc

_NN = 13


def kernel(num_features, cat_features, tables):
    N = num_features.shape[0]
    F, V, D = tables.shape
    d_out = _NN + F * D

    tab = tables.reshape(F * V, D)
    idx = cat_features.astype(jnp.int32) + jnp.arange(F, dtype=jnp.int32) * V
    idxg = idx.reshape(N * F // 128, 128)
    num_flat = num_features.reshape(N * _NN)

    NW = 32
    RW = N // NW
    RC = 64
    NCH = RW // RC
    GB = RC * F // 128

    mesh = plsc.VectorSubcoreMesh(core_axis_name="c", subcore_axis_name="s")

    @functools.partial(
        pl.kernel,
        out_type=jax.ShapeDtypeStruct((N * d_out,), jnp.float32),
        mesh=mesh,
        scratch_types=[
            pltpu.VMEM((GB, 128), jnp.int32),
            pltpu.VMEM((RC * F, D), jnp.float32),
            pltpu.VMEM((RC * _NN + 8,), jnp.float32),
            pltpu.VMEM((RC * d_out,), jnp.float32),
            pltpu.SemaphoreType.DMA,
        ],
        compiler_params=pltpu.CompilerParams(use_tc_tiling_on_sc=False),
    )
    def _embed(tab_hbm, idxg_hbm, num_hbm, out_hbm,
               idx_v, emb_v, num_v, out_c, sem):
        wid = lax.axis_index("s") * 2 + lax.axis_index("c")
        w_r0 = wid * RW

        def chunk_body(c, _):
            r0 = w_r0 + c * RC
            pltpu.sync_copy(idxg_hbm.at[pl.ds(r0 * F // 128, GB)], idx_v)
            pltpu.sync_copy(
                num_hbm.at[pl.ds(r0 * _NN, RC * _NN)],
                num_v.at[pl.ds(0, RC * _NN)],
            )
            copies = [
                pltpu.async_copy(
                    tab_hbm.at[idx_v.at[j]],
                    emb_v.at[pl.ds(j * 128, 128), :],
                    sem,
                )
                for j in range(GB)
            ]
            for cp in copies:
                cp.wait()

            def row_body(r, _):
                ooff = r * d_out
                out_c[pl.ds(ooff, 16)] = num_v[pl.ds(r * _NN, 16)]
                for k in range(F):
                    out_c[pl.ds(ooff + _NN + k * D, 16)] = emb_v[r * F + k, :]
                return 0

            lax.fori_loop(0, RC, row_body, 0, unroll=False)
            pltpu.sync_copy(out_c, out_hbm.at[pl.ds(r0 * d_out, RC * d_out)])
            return 0

        lax.fori_loop(0, NCH, chunk_body, 0, unroll=False)

    out_flat = _embed(tab, idxg, num_flat)
    return out_flat.reshape(N, d_out)
